# Initial kernel scaffold; baseline (speedup 1.0000x reference)
import jax
import jax.numpy as jnp
from jax.experimental import pallas as pl


def kernel(x, edge_index, W1_self, W1_neigh, b1, W2_self, W2_neigh, b2, W3_self, W3_neigh, b3):
    raise NotImplementedError("write your pallas kernel here")


# speedup vs baseline: 4.4599x; 4.4599x over previous
import functools

import jax
import jax.numpy as jnp
from jax import lax
from jax.experimental import pallas as pl
from jax.experimental.pallas import tpu as pltpu
from jax.experimental.pallas import tpu_sc as plsc

_NC = 2
_NS = 16
_K = 128


def _make_sc_agg(n_nodes, n_edges, dh):
    assert n_edges % _NS == 0
    e_tile = n_edges // _NS
    nfull = e_tile // _K
    rem = e_tile % _K
    rows_tile = -(-(n_nodes // _NS) // 8) * 8
    rows_last = n_nodes - (_NS - 1) * rows_tile
    assert 0 < rows_last <= rows_tile
    assert dh % 16 == 0

    mesh = plsc.VectorSubcoreMesh(
        core_axis_name="c", subcore_axis_name="s",
        num_cores=_NC, num_subcores=_NS)

    scratch = [
        pltpu.VMEM((_K,), jnp.int32),
        pltpu.VMEM((_K,), jnp.int32),
        pltpu.VMEM((_K, dh), jnp.float32),
        pltpu.VMEM_SHARED((n_nodes, dh), jnp.float32),
        pltpu.SemaphoreType.DMA,
    ]
    if rem:
        scratch = scratch[:3] + [
            pltpu.VMEM((rem,), jnp.int32),
            pltpu.VMEM((rem,), jnp.int32),
            pltpu.VMEM((rem, dh), jnp.float32),
        ] + scratch[3:]

    @functools.partial(
        pl.kernel,
        out_type=jax.ShapeDtypeStruct((_NC, n_nodes, dh), jnp.float32),
        mesh=mesh,
        scratch_types=scratch,
        compiler_params=pltpu.CompilerParams(use_tc_tiling_on_sc=False),
    )
    def agg_kernel(t_hbm, src_hbm, dst_hbm, z_hbm, out_hbm, *scr):
        if rem:
            sidx, didx, rows, sidx2, didx2, rows2, acc, sem = scr
        else:
            sidx, didx, rows, acc, sem = scr
        c = lax.axis_index("c")
        s = lax.axis_index("s")
        tbl = t_hbm.at[c]

        r0 = s * rows_tile

        @pl.when(s < _NS - 1)
        def _():
            pltpu.sync_copy(z_hbm.at[pl.ds(r0, rows_tile)],
                            acc.at[pl.ds(r0, rows_tile)])

        @pl.when(s == _NS - 1)
        def _():
            pltpu.sync_copy(z_hbm.at[pl.ds(r0, rows_last)],
                            acc.at[pl.ds(r0, rows_last)])

        plsc.subcore_barrier()

        base = s * e_tile

        @pl.loop(0, nfull)
        def _(g):
            off = base + g * _K
            pltpu.sync_copy(src_hbm.at[pl.ds(off, _K)], sidx)
            pltpu.sync_copy(dst_hbm.at[pl.ds(off, _K)], didx)
            pltpu.async_copy(tbl.at[sidx], rows, sem).wait()
            pltpu.sync_copy(rows, acc.at[didx], add=True)

        if rem:
            off = base + nfull * _K
            pltpu.sync_copy(src_hbm.at[pl.ds(off, rem)], sidx2)
            pltpu.sync_copy(dst_hbm.at[pl.ds(off, rem)], didx2)
            pltpu.async_copy(tbl.at[sidx2], rows2, sem).wait()
            pltpu.sync_copy(rows2, acc.at[didx2], add=True)

        plsc.subcore_barrier()

        @pl.when(s < _NS - 1)
        def _():
            pltpu.sync_copy(acc.at[pl.ds(r0, rows_tile)],
                            out_hbm.at[c, pl.ds(r0, rows_tile)])

        @pl.when(s == _NS - 1)
        def _():
            pltpu.sync_copy(acc.at[pl.ds(r0, rows_last)],
                            out_hbm.at[c, pl.ds(r0, rows_last)])

    return agg_kernel


_BM = 400


def _dot(a, b):
    return jnp.dot(a, b, preferred_element_type=jnp.float32)


def _layer1_tc(x, agg1, W1s, W1na, W1nb, b1r, n_nodes, dh_in, dh):
    grid = n_nodes // _BM

    def body(x_ref, a_ref, ws_ref, wna_ref, wnb_ref, b_ref, h_ref, rdeg_ref):
        deg = a_ref[0][:, 64:65]
        r = 1.0 / jnp.maximum(deg, 1.0)
        neigh = (_dot(a_ref[0], wna_ref[...]) + _dot(a_ref[1], wnb_ref[...])) * r
        h = jax.nn.relu(_dot(x_ref[...], ws_ref[...]) + neigh + b_ref[...])
        h_ref[0] = h[:, :128]
        h_ref[1] = h[:, 128:]
        rdeg_ref[...] = r

    return pl.pallas_call(
        body,
        grid=(grid,),
        in_specs=[
            pl.BlockSpec((_BM, dh_in), lambda m: (m, 0)),
            pl.BlockSpec((2, _BM, 80), lambda m: (0, m, 0)),
            pl.BlockSpec((dh_in, 256), lambda m: (0, 0)),
            pl.BlockSpec((80, 256), lambda m: (0, 0)),
            pl.BlockSpec((80, 256), lambda m: (0, 0)),
            pl.BlockSpec((1, 256), lambda m: (0, 0)),
        ],
        out_specs=[
            pl.BlockSpec((2, _BM, 128), lambda m: (0, m, 0)),
            pl.BlockSpec((_BM, 1), lambda m: (m, 0)),
        ],
        out_shape=[
            jax.ShapeDtypeStruct((2, n_nodes, 128), jnp.float32),
            jax.ShapeDtypeStruct((n_nodes, 1), jnp.float32),
        ],
    )(x, agg1, W1s, W1na, W1nb, b1r)


def _layer2_tc(h1s, agg2, rdeg, W2sa, W2sb, W2na, W2nb, b2r, W3np, n_nodes):
    grid = n_nodes // _BM

    def body(h_ref, a_ref, r_ref, wsa, wsb, wna, wnb, b_ref, w3n, h2_ref, t3_ref):
        r = r_ref[...]
        neigh = (_dot(a_ref[0], wna[...]) + _dot(a_ref[1], wnb[...])) * r
        h2 = jax.nn.relu(_dot(h_ref[0], wsa[...]) + _dot(h_ref[1], wsb[...])
                         + neigh + b_ref[...])
        t3 = _dot(h2, w3n[...])
        h2_ref[0] = h2[:, :128]
        h2_ref[1] = h2[:, 128:]
        t3_ref[0] = t3[:, :32]
        t3_ref[1] = t3[:, 32:]

    return pl.pallas_call(
        body,
        grid=(grid,),
        in_specs=[
            pl.BlockSpec((2, _BM, 128), lambda m: (0, m, 0)),
            pl.BlockSpec((2, _BM, 128), lambda m: (0, m, 0)),
            pl.BlockSpec((_BM, 1), lambda m: (m, 0)),
            pl.BlockSpec((128, 256), lambda m: (0, 0)),
            pl.BlockSpec((128, 256), lambda m: (0, 0)),
            pl.BlockSpec((128, 256), lambda m: (0, 0)),
            pl.BlockSpec((128, 256), lambda m: (0, 0)),
            pl.BlockSpec((1, 256), lambda m: (0, 0)),
            pl.BlockSpec((256, 64), lambda m: (0, 0)),
        ],
        out_specs=[
            pl.BlockSpec((2, _BM, 128), lambda m: (0, m, 0)),
            pl.BlockSpec((2, _BM, 32), lambda m: (0, m, 0)),
        ],
        out_shape=[
            jax.ShapeDtypeStruct((2, n_nodes, 128), jnp.float32),
            jax.ShapeDtypeStruct((2, n_nodes, 32), jnp.float32),
        ],
    )(h1s, agg2, rdeg, W2sa, W2sb, W2na, W2nb, b2r, W3np)


def _layer3_tc(h2s, agg3, rdeg, W3sa, W3sb, b3p, n_nodes):
    grid = n_nodes // _BM

    def body(h_ref, a_ref, r_ref, wsa, wsb, b_ref, o_ref):
        r = r_ref[...]
        neigh = jnp.concatenate([a_ref[0], a_ref[1]], axis=1) * r
        o_ref[...] = (_dot(h_ref[0], wsa[...]) + _dot(h_ref[1], wsb[...])
                      + neigh + b_ref[...])

    return pl.pallas_call(
        body,
        grid=(grid,),
        in_specs=[
            pl.BlockSpec((2, _BM, 128), lambda m: (0, m, 0)),
            pl.BlockSpec((2, _BM, 32), lambda m: (0, m, 0)),
            pl.BlockSpec((_BM, 1), lambda m: (m, 0)),
            pl.BlockSpec((128, 64), lambda m: (0, 0)),
            pl.BlockSpec((128, 64), lambda m: (0, 0)),
            pl.BlockSpec((1, 64), lambda m: (0, 0)),
        ],
        out_specs=pl.BlockSpec((_BM, 64), lambda m: (m, 0)),
        out_shape=jax.ShapeDtypeStruct((n_nodes, 64), jnp.float32),
    )(h2s, agg3, rdeg, W3sa, W3sb, b3p)


def kernel(x, edge_index, W1_self, W1_neigh, b1, W2_self, W2_neigh, b2,
           W3_self, W3_neigh, b3):
    n_nodes, d_in = x.shape
    n_edges = edge_index.shape[1]
    d_out = W3_self.shape[1]
    src = edge_index[0]
    dst = edge_index[1]

    ones = jnp.ones((n_nodes, 1), jnp.float32)
    zpad = jnp.zeros((n_nodes, 15), jnp.float32)
    x0 = jnp.concatenate([x[:, :64], ones, zpad], axis=1)
    x1 = jnp.concatenate([x[:, 64:], zpad, jnp.zeros((n_nodes, 1), jnp.float32)],
                         axis=1)
    xs = jnp.stack([x0, x1])

    z80 = jnp.zeros((n_nodes, 80), jnp.float32)
    z128 = jnp.zeros((n_nodes, 128), jnp.float32)
    z32 = jnp.zeros((n_nodes, 32), jnp.float32)

    wpad16 = jnp.zeros((16, 256), jnp.float32)
    W1na = jnp.concatenate([W1_neigh[:64], wpad16], axis=0)
    W1nb = jnp.concatenate([W1_neigh[64:], wpad16], axis=0)
    b1r = b1.reshape(1, -1)
    W2sa, W2sb = W2_self[:128], W2_self[128:]
    W2na, W2nb = W2_neigh[:128], W2_neigh[128:]
    b2r = b2.reshape(1, -1)
    cpad = jnp.zeros((256, 64 - d_out), jnp.float32)
    W3np = jnp.concatenate([W3_neigh, cpad], axis=1)
    W3sp = jnp.concatenate([W3_self, cpad], axis=1)
    W3sa, W3sb = W3sp[:128], W3sp[128:]
    b3p = jnp.concatenate([b3, jnp.zeros((64 - d_out,), jnp.float32)]
                          ).reshape(1, -1)

    agg1 = _make_sc_agg(n_nodes, n_edges, 80)(xs, src, dst, z80)
    h1s, rdeg = _layer1_tc(x, agg1, W1_self, W1na, W1nb, b1r,
                           n_nodes, d_in, 256)
    agg2 = _make_sc_agg(n_nodes, n_edges, 128)(h1s, src, dst, z128)
    h2s, t3s = _layer2_tc(h1s, agg2, rdeg, W2sa, W2sb, W2na, W2nb, b2r,
                          W3np, n_nodes)
    agg3 = _make_sc_agg(n_nodes, n_edges, 32)(t3s, src, dst, z32)
    outp = _layer3_tc(h2s, agg3, rdeg, W3sa, W3sb, b3p, n_nodes)
    return outp[:, :d_out]
